# baseline (speedup 1.0000x reference)
import functools

import jax
import jax.numpy as jnp
from jax import lax
from jax.experimental import pallas as pl
from jax.experimental.pallas import tpu as pltpu
from jax.experimental.pallas import tpu_sc as plsc

_B, _K, _D = 4096, 8192, 32
_BB = 256
_NW = 32
_BPW = _B // _NW


def _sc_body(cb_hbm, codes_hbm, lflat_hbm, fidx_hbm, hard_hbm, lcode_hbm,
             idx_v, rows_v, fidx_v, vals_v, sem_a, sem_b):
    wid = lax.axis_index("s") * 2 + lax.axis_index("c")
    base = wid * _BPW
    pltpu.sync_copy(codes_hbm.at[pl.ds(base, _BPW)], idx_v)
    pltpu.sync_copy(fidx_hbm.at[pl.ds(base, _BPW)], fidx_v)
    cp_rows = pltpu.async_copy(cb_hbm.at[idx_v], rows_v, sem_a)
    cp_vals = pltpu.async_copy(lflat_hbm.at[fidx_v], vals_v, sem_b)
    cp_rows.wait()
    cp_vals.wait()
    pltpu.sync_copy(rows_v, hard_hbm.at[pl.ds(base, _BPW)])
    pltpu.sync_copy(vals_v, lcode_hbm.at[pl.ds(base, _BPW)])


_sc_gather = pl.kernel(
    _sc_body,
    out_type=[
        jax.ShapeDtypeStruct((_B, _D), jnp.float32),
        jax.ShapeDtypeStruct((_B,), jnp.float32),
    ],
    mesh=plsc.VectorSubcoreMesh(core_axis_name="c", subcore_axis_name="s"),
    scratch_types=[
        pltpu.VMEM((_BPW,), jnp.int32),
        pltpu.VMEM((_BPW, _D), jnp.float32),
        pltpu.VMEM((_BPW,), jnp.int32),
        pltpu.VMEM((_BPW,), jnp.float32),
        pltpu.SemaphoreType.DMA,
        pltpu.SemaphoreType.DMA,
    ],
    compiler_params=pltpu.CompilerParams(use_tc_tiling_on_sc=False),
)


def _tc_body(logits_ref, cb_ref, hard_ref, lcode_ref, out_ref):
    x = logits_ref[...]
    m = jnp.max(x, axis=1, keepdims=True)
    e = jnp.exp(x - m)
    t = jnp.sum(e * x, axis=1, keepdims=True)

    dn = (((1,), (0,)), ((), ()))
    vs = lax.dot_general(e.astype(jnp.bfloat16), cb_ref[...], dn,
                         preferred_element_type=jnp.float32)
    v = vs[:, :_D]
    s = vs[:, _D:]
    logs = jnp.log(s)

    out_ref[...] = jnp.concatenate(
        [hard_ref[...], v / s, lcode_ref[...] - m - logs, m + logs - t / s],
        axis=1)


@jax.jit
def kernel(logits, codes, codebook):
    fidx = jnp.arange(_B, dtype=jnp.int32) * _K + codes
    hard, lcode = _sc_gather(codebook, codes, logits.reshape(_B * _K), fidx)
    cb_ext = jnp.concatenate(
        [codebook.astype(jnp.bfloat16),
         jnp.ones((_K, 1), jnp.bfloat16)], axis=1)
    return pl.pallas_call(
        _tc_body,
        grid=(_B // _BB,),
        in_specs=[
            pl.BlockSpec((_BB, _K), lambda i: (i, 0)),
            pl.BlockSpec((_K, _D + 1), lambda i: (0, 0)),
            pl.BlockSpec((_BB, _D), lambda i: (i, 0)),
            pl.BlockSpec((_BB, 1), lambda i: (i, 0)),
        ],
        out_specs=pl.BlockSpec((_BB, 2 * _D + 2), lambda i: (i, 0)),
        out_shape=jax.ShapeDtypeStruct((_B, 2 * _D + 2), jnp.float32),
    )(logits, cb_ext, hard, lcode.reshape(_B, 1))


# speedup vs baseline: 1.8426x; 1.8426x over previous
import functools

import jax
import jax.numpy as jnp
from jax import lax
from jax.experimental import pallas as pl
from jax.experimental.pallas import tpu as pltpu
from jax.experimental.pallas import tpu_sc as plsc

_B, _K, _D = 4096, 8192, 32
_BB = 256
_NW = 32
_BPW = _B // _NW


def _sc_body(cb_hbm, codes_hbm, hard_hbm, idx_v, rows_v, sem):
    wid = lax.axis_index("s") * 2 + lax.axis_index("c")
    base = wid * _BPW
    pltpu.sync_copy(codes_hbm.at[pl.ds(base, _BPW)], idx_v)
    pltpu.async_copy(cb_hbm.at[idx_v], rows_v, sem).wait()
    pltpu.sync_copy(rows_v, hard_hbm.at[pl.ds(base, _BPW)])


_sc_gather = pl.kernel(
    _sc_body,
    out_type=jax.ShapeDtypeStruct((_B, _D), jnp.float32),
    mesh=plsc.VectorSubcoreMesh(core_axis_name="c", subcore_axis_name="s"),
    scratch_types=[
        pltpu.VMEM((_BPW,), jnp.int32),
        pltpu.VMEM((_BPW, _D), jnp.float32),
        pltpu.SemaphoreType.DMA,
    ],
    compiler_params=pltpu.CompilerParams(use_tc_tiling_on_sc=False),
)


def _tc_body(logits_ref, codes_ref, cb_ref, hard_ref, out_ref):
    x = logits_ref[...]
    m = jnp.max(x, axis=1, keepdims=True)
    e = jnp.exp(x - m)
    t = jnp.sum(e * x, axis=1, keepdims=True)

    codes = codes_ref[...]
    iota = lax.broadcasted_iota(jnp.int32, (_BB, _K), 1)
    oh = iota == codes
    l_code = jnp.sum(jnp.where(oh, x, 0.0), axis=1, keepdims=True)

    dn = (((1,), (0,)), ((), ()))
    vs = lax.dot_general(e.astype(jnp.bfloat16), cb_ref[...], dn,
                         preferred_element_type=jnp.float32)
    v = vs[:, :_D]
    s = vs[:, _D:]
    logs = jnp.log(s)

    out_ref[...] = jnp.concatenate(
        [hard_ref[...], v / s, l_code - m - logs, m + logs - t / s], axis=1)


@jax.jit
def kernel(logits, codes, codebook):
    hard = _sc_gather(codebook, codes)
    cb_ext = jnp.concatenate(
        [codebook.astype(jnp.bfloat16),
         jnp.ones((_K, 1), jnp.bfloat16)], axis=1)
    return pl.pallas_call(
        _tc_body,
        grid=(_B // _BB,),
        in_specs=[
            pl.BlockSpec((_BB, _K), lambda i: (i, 0)),
            pl.BlockSpec((_BB, 1), lambda i: (i, 0)),
            pl.BlockSpec((_K, _D + 1), lambda i: (0, 0)),
            pl.BlockSpec((_BB, _D), lambda i: (i, 0)),
        ],
        out_specs=pl.BlockSpec((_BB, 2 * _D + 2), lambda i: (i, 0)),
        out_shape=jax.ShapeDtypeStruct((_B, 2 * _D + 2), jnp.float32),
    )(logits, codes.reshape(_B, 1), cb_ext, hard)


# speedup vs baseline: 1.9874x; 1.0786x over previous
import functools

import jax
import jax.numpy as jnp
from jax import lax
from jax.experimental import pallas as pl
from jax.experimental.pallas import tpu as pltpu
from jax.experimental.pallas import tpu_sc as plsc

_B, _K, _D = 4096, 8192, 32
_BB = 256
_NW = 32
_BPW = _B // _NW


def _sc_body(cb_hbm, codes_hbm, hard_hbm, idx_v, rows_v, sem):
    wid = lax.axis_index("s") * 2 + lax.axis_index("c")
    base = wid * _BPW
    pltpu.sync_copy(codes_hbm.at[pl.ds(base, _BPW)], idx_v)
    pltpu.async_copy(cb_hbm.at[idx_v], rows_v, sem).wait()
    pltpu.sync_copy(rows_v, hard_hbm.at[pl.ds(base, _BPW)])


_sc_gather = pl.kernel(
    _sc_body,
    out_type=jax.ShapeDtypeStruct((_B, _D), jnp.float32),
    mesh=plsc.VectorSubcoreMesh(core_axis_name="c", subcore_axis_name="s"),
    scratch_types=[
        pltpu.VMEM((_BPW,), jnp.int32),
        pltpu.VMEM((_BPW, _D), jnp.float32),
        pltpu.SemaphoreType.DMA,
    ],
    compiler_params=pltpu.CompilerParams(use_tc_tiling_on_sc=False),
)


def _tc_body(logits_ref, codes_ref, cb_ref, out_ref):
    x = logits_ref[...]
    m = jnp.max(x, axis=1, keepdims=True)
    e = jnp.exp(x - m)
    t = jnp.sum(e * x, axis=1, keepdims=True)

    codes = codes_ref[...]
    iota = lax.broadcasted_iota(jnp.int32, (_BB, _K), 1)
    oh = iota == codes
    l_code = jnp.sum(jnp.where(oh, x, 0.0), axis=1, keepdims=True)

    dn = (((1,), (0,)), ((), ()))
    cb = cb_ref[...]
    vs = lax.dot_general(e.astype(jnp.bfloat16), cb, dn,
                         preferred_element_type=jnp.float32)
    hard = lax.dot_general(oh.astype(jnp.bfloat16), cb, dn,
                           preferred_element_type=jnp.float32)[:, :_D]
    v = vs[:, :_D]
    s = vs[:, _D:]
    logs = jnp.log(s)

    out_ref[...] = jnp.concatenate(
        [hard, v / s, l_code - m - logs, m + logs - t / s], axis=1)


@jax.jit
def kernel(logits, codes, codebook):
    cb_ext = jnp.concatenate(
        [codebook.astype(jnp.bfloat16),
         jnp.ones((_K, 1), jnp.bfloat16)], axis=1)
    return pl.pallas_call(
        _tc_body,
        grid=(_B // _BB,),
        in_specs=[
            pl.BlockSpec((_BB, _K), lambda i: (i, 0)),
            pl.BlockSpec((_BB, 1), lambda i: (i, 0)),
            pl.BlockSpec((_K, _D + 1), lambda i: (0, 0)),
        ],
        out_specs=pl.BlockSpec((_BB, 2 * _D + 2), lambda i: (i, 0)),
        out_shape=jax.ShapeDtypeStruct((_B, 2 * _D + 2), jnp.float32),
    )(logits, codes.reshape(_B, 1), cb_ext)
